# baseline (speedup 1.0000x reference)
import functools

import jax
import jax.numpy as jnp
from jax import lax
from jax.experimental import pallas as pl
from jax.experimental.pallas import tpu as pltpu
from jax.experimental.pallas import tpu_sc as plsc

NC = 2
NS = 16
LANES = 16
K = 128
HC = 8
S0 = 20
S1 = 0


def _sc_mesh():
    return plsc.VectorSubcoreMesh(core_axis_name="c", subcore_axis_name="s")


@functools.cache
def _hist_kernel(E, N, NL):
    per_w = E // (NC * NS)
    n16 = NL // LANES
    e16 = per_w // LANES

    @functools.partial(
        pl.kernel,
        out_type=jax.ShapeDtypeStruct((NC * NS, NL), jnp.float32),
        mesh=_sc_mesh(),
        scratch_types=[
            pltpu.VMEM((per_w,), jnp.int32),
            pltpu.VMEM((NL,), jnp.float32),
        ],
        compiler_params=pltpu.CompilerParams(needs_layout_passes=False),
    )
    def hist(dst_hbm, out_hbm, idx_v, hist_v):
        c = lax.axis_index("c")
        s = lax.axis_index("s")
        w = c * NS + s

        def zero_body(i, carry):
            hist_v[pl.ds(i * LANES, LANES)] = jnp.zeros((LANES,), jnp.float32)
            return carry

        lax.fori_loop(0, n16, zero_body, 0)

        pltpu.sync_copy(dst_hbm.at[pl.ds(w * per_w, per_w)], idx_v)
        ones = jnp.ones((LANES,), jnp.float32)

        def acc_body(i, carry):
            idxs = idx_v[pl.ds(i * LANES, LANES)]
            plsc.addupdate_scatter(hist_v, [idxs], ones)
            return carry

        lax.fori_loop(0, e16, acc_body, 0)

        pltpu.sync_copy(hist_v, out_hbm.at[w])

    return hist


@functools.cache
def _agg_kernel(n_slab, hc, N, D):
    NROWS = 10240
    STR = NROWS // NS

    @functools.partial(
        pl.kernel,
        out_type=jax.ShapeDtypeStruct((NC, N, D), jnp.float32),
        mesh=_sc_mesh(),
        scratch_types=[
            pltpu.VMEM((2, hc, K), jnp.int32),
            pltpu.VMEM((2, hc, K), jnp.int32),
            pltpu.VMEM((K, D), jnp.float32),
            pltpu.VMEM((K, D), jnp.float32),
            pltpu.VMEM_SHARED((NROWS, D), jnp.float32),
            pltpu.SemaphoreType.DMA,
            pltpu.SemaphoreType.DMA,
            pltpu.SemaphoreType.DMA,
        ],
    )
    def agg(g_hbm, zeros_hbm, src_hbm, dst_hbm, out_hbm,
            sidx2, didx2, rows0, rows1, acc_s, sem0, sem1, isem):
        c = lax.axis_index("c")
        s = lax.axis_index("s")

        pltpu.sync_copy(zeros_hbm, acc_s.at[pl.ds(s * STR, STR)])
        plsc.subcore_barrier()

        my_slabs = jnp.where(c == 0, S0, S1)
        slab_base = jnp.where(c == 0, s * S0, NS * S0 + s * S1)

        def load_slab_p(h, par):
            slab = slab_base + h
            pltpu.async_copy(src_hbm.at[slab], sidx2.at[par], isem)
            pltpu.async_copy(dst_hbm.at[slab], didx2.at[par], isem)

        def wait_slab_p(h, par):
            slab = slab_base + h
            pltpu.make_async_copy(src_hbm.at[slab], sidx2.at[par], isem).wait()
            pltpu.make_async_copy(dst_hbm.at[slab], didx2.at[par], isem).wait()

        @pl.when(my_slabs > 0)
        def _():
            pltpu.sync_copy(src_hbm.at[slab_base], sidx2.at[0])
            pltpu.sync_copy(dst_hbm.at[slab_base], didx2.at[0])

        def do_slab(h, par):
            sidx, didx = sidx2.at[par], didx2.at[par]

            @pl.when(h > 0)
            def _():
                wait_slab_p(h, par)

            @pl.when(h + 1 < my_slabs)
            def _():
                load_slab_p(h + 1, 1 - par)

            pltpu.async_copy(g_hbm.at[sidx.at[0]], rows0, sem0)

            def pair_body(i, carry):
                e0 = 2 * i
                e1 = e0 + 1
                e2 = e0 + 2
                pltpu.async_copy(g_hbm.at[sidx.at[e1]], rows1, sem1)
                pltpu.make_async_copy(g_hbm.at[sidx.at[e0]], rows0, sem0).wait()
                pltpu.sync_copy(rows0, acc_s.at[didx.at[e0]], add=True)

                @pl.when(e2 < hc)
                def _():
                    pltpu.async_copy(g_hbm.at[sidx.at[e2]], rows0, sem0)

                pltpu.make_async_copy(g_hbm.at[sidx.at[e1]], rows1, sem1).wait()
                pltpu.sync_copy(rows1, acc_s.at[didx.at[e1]], add=True)
                return carry

            lax.fori_loop(0, hc // 2, pair_body, 0)

        def slab_pair(ip, carry):
            for par in (0, 1):
                h = 2 * ip + par

                @pl.when(h < my_slabs)
                def _(h=h, par=par):
                    do_slab(h, par)
            return carry

        lax.fori_loop(0, (max(S0, S1) + 1) // 2, slab_pair, 0)
        plsc.subcore_barrier()

        @pl.when(s < NS - 1)
        def _():
            pltpu.sync_copy(acc_s.at[pl.ds(s * STR, STR)],
                            out_hbm.at[c, pl.ds(s * STR, STR)])

        last = N - (NS - 1) * STR

        @pl.when(s == NS - 1)
        def _():
            pltpu.sync_copy(acc_s.at[pl.ds((NS - 1) * STR, last)],
                            out_hbm.at[c, pl.ds((NS - 1) * STR, last)])

    return agg


@functools.cache
def _prescale_call(N, D, NL):
    R = 1024
    G = NL // R

    def body(h_ref, x_ref, dinv_ref, g_ref):
        ones = jnp.ones((NC * NS, 1), jnp.float32)
        deg = lax.dot_general(h_ref[...], ones, (((0,), (0,)), ((), ())),
                              preferred_element_type=jnp.float32,
                              precision=lax.Precision.HIGHEST) + 1.0
        dv = lax.rsqrt(deg)
        dinv_ref[...] = dv
        g_ref[...] = dv * x_ref[...]

    return pl.pallas_call(
        body,
        grid=(G,),
        in_specs=[
            pl.BlockSpec((NC * NS, R), lambda i: (0, i)),
            pl.BlockSpec((R, D), lambda i: (i, 0)),
        ],
        out_specs=[
            pl.BlockSpec((R, 1), lambda i: (i, 0)),
            pl.BlockSpec((R, D), lambda i: (i, 0)),
        ],
        out_shape=[
            jax.ShapeDtypeStruct((NL, 1), jnp.float32),
            jax.ShapeDtypeStruct((N, D), jnp.float32),
        ],
    )


@functools.cache
def _finalize_call(N, D):
    R = 400
    G = N // R

    def body(x_ref, g_ref, acc_ref, dv_ref, w_ref, b_ref, o_ref):
        S = (acc_ref[0, :, :] + acc_ref[1, :, :] + g_ref[...]) * dv_ref[...]
        T = jnp.dot(S, w_ref[...], preferred_element_type=jnp.float32,
                    precision=lax.Precision.HIGHEST)
        o_ref[...] = x_ref[...] + jnp.maximum(T + b_ref[...], 0.0)

    return pl.pallas_call(
        body,
        grid=(G,),
        in_specs=[
            pl.BlockSpec((R, D), lambda i: (i, 0)),
            pl.BlockSpec((R, D), lambda i: (i, 0)),
            pl.BlockSpec((NC, R, D), lambda i: (0, i, 0)),
            pl.BlockSpec((R, 1), lambda i: (i, 0)),
            pl.BlockSpec((D, D), lambda i: (0, 0)),
            pl.BlockSpec((1, D), lambda i: (0, 0)),
        ],
        out_specs=pl.BlockSpec((R, D), lambda i: (i, 0)),
        out_shape=jax.ShapeDtypeStruct((N, D), jnp.float32),
    )


@jax.jit
def kernel(x, edge_index, W, b):
    N, D = x.shape
    E = edge_index.shape[1]
    src = edge_index[0]
    dst = edge_index[1]

    NL = 10240
    hist = _hist_kernel(E, N, NL)(dst)
    dinv, g = _prescale_call(N, D, NL)(hist, x)

    n_slab = NS * (S0 + S1)
    pad = n_slab * HC * K - E
    assert pad >= 0
    src_p = jnp.concatenate([src, jnp.zeros((pad,), src.dtype)])
    dst_p = jnp.concatenate([dst, jnp.full((pad,), N, dst.dtype)])
    zeros = jnp.zeros((10240 // NS, D), jnp.float32)
    acc = _agg_kernel(n_slab, HC, N, D)(
        g, zeros, src_p.reshape(n_slab, HC, K),
        dst_p.reshape(n_slab, HC, K))
    return _finalize_call(N, D)(x, g, acc, dinv, W, b.reshape(1, D))


# speedup vs baseline: 1.2326x; 1.2326x over previous
import functools

import jax
import jax.numpy as jnp
from jax import lax
from jax.experimental import pallas as pl
from jax.experimental.pallas import tpu as pltpu
from jax.experimental.pallas import tpu_sc as plsc

NC = 2
NS = 16
LANES = 16
K = 128
HC = 8
S0 = 17
S1 = 3


def _sc_mesh():
    return plsc.VectorSubcoreMesh(core_axis_name="c", subcore_axis_name="s")


@functools.cache
def _hist_kernel(E, N, NL):
    per_w = E // (NC * NS)
    n16 = NL // LANES
    e16 = per_w // LANES

    @functools.partial(
        pl.kernel,
        out_type=jax.ShapeDtypeStruct((NC * NS, NL), jnp.float32),
        mesh=_sc_mesh(),
        scratch_types=[
            pltpu.VMEM((per_w,), jnp.int32),
            pltpu.VMEM((NL,), jnp.float32),
        ],
        compiler_params=pltpu.CompilerParams(needs_layout_passes=False),
    )
    def hist(dst_hbm, out_hbm, idx_v, hist_v):
        c = lax.axis_index("c")
        s = lax.axis_index("s")
        w = c * NS + s

        def zero_body(i, carry):
            hist_v[pl.ds(i * LANES, LANES)] = jnp.zeros((LANES,), jnp.float32)
            return carry

        lax.fori_loop(0, n16, zero_body, 0)

        pltpu.sync_copy(dst_hbm.at[pl.ds(w * per_w, per_w)], idx_v)
        ones = jnp.ones((LANES,), jnp.float32)

        def acc_body(i, carry):
            idxs = idx_v[pl.ds(i * LANES, LANES)]
            plsc.addupdate_scatter(hist_v, [idxs], ones)
            return carry

        lax.fori_loop(0, e16, acc_body, 0)

        pltpu.sync_copy(hist_v, out_hbm.at[w])

    return hist


@functools.cache
def _agg_kernel(n_slab, hc, N, D):
    NROWS = 10240
    STR = NROWS // NS

    @functools.partial(
        pl.kernel,
        out_type=jax.ShapeDtypeStruct((NC, N, D), jnp.float32),
        mesh=_sc_mesh(),
        scratch_types=[
            pltpu.VMEM((2, hc, K), jnp.int32),
            pltpu.VMEM((2, hc, K), jnp.int32),
            pltpu.VMEM((K, D), jnp.float32),
            pltpu.VMEM((K, D), jnp.float32),
            pltpu.VMEM_SHARED((NROWS, D), jnp.float32),
            pltpu.SemaphoreType.DMA,
            pltpu.SemaphoreType.DMA,
            pltpu.SemaphoreType.DMA,
        ],
    )
    def agg(g_hbm, zeros_hbm, src_hbm, dst_hbm, out_hbm,
            sidx2, didx2, rows0, rows1, acc_s, sem0, sem1, isem):
        c = lax.axis_index("c")
        s = lax.axis_index("s")

        pltpu.sync_copy(zeros_hbm, acc_s.at[pl.ds(s * STR, STR)])
        plsc.subcore_barrier()

        my_slabs = jnp.where(c == 0, S0, S1)
        slab_base = jnp.where(c == 0, s * S0, NS * S0 + s * S1)

        def load_slab_p(h, par):
            slab = slab_base + h
            pltpu.async_copy(src_hbm.at[slab], sidx2.at[par], isem)
            pltpu.async_copy(dst_hbm.at[slab], didx2.at[par], isem)

        def wait_slab_p(h, par):
            slab = slab_base + h
            pltpu.make_async_copy(src_hbm.at[slab], sidx2.at[par], isem).wait()
            pltpu.make_async_copy(dst_hbm.at[slab], didx2.at[par], isem).wait()

        @pl.when(my_slabs > 0)
        def _():
            pltpu.sync_copy(src_hbm.at[slab_base], sidx2.at[0])
            pltpu.sync_copy(dst_hbm.at[slab_base], didx2.at[0])

        def do_slab(h, par):
            sidx, didx = sidx2.at[par], didx2.at[par]

            @pl.when(h > 0)
            def _():
                wait_slab_p(h, par)

            @pl.when(h + 1 < my_slabs)
            def _():
                load_slab_p(h + 1, 1 - par)

            pltpu.async_copy(g_hbm.at[sidx.at[0]], rows0, sem0)

            def pair_body(i, carry):
                e0 = 2 * i
                e1 = e0 + 1
                e2 = e0 + 2
                pltpu.async_copy(g_hbm.at[sidx.at[e1]], rows1, sem1)
                pltpu.make_async_copy(g_hbm.at[sidx.at[e0]], rows0, sem0).wait()
                pltpu.sync_copy(rows0, acc_s.at[didx.at[e0]], add=True)

                @pl.when(e2 < hc)
                def _():
                    pltpu.async_copy(g_hbm.at[sidx.at[e2]], rows0, sem0)

                pltpu.make_async_copy(g_hbm.at[sidx.at[e1]], rows1, sem1).wait()
                pltpu.sync_copy(rows1, acc_s.at[didx.at[e1]], add=True)
                return carry

            lax.fori_loop(0, hc // 2, pair_body, 0)

        def slab_pair(ip, carry):
            for par in (0, 1):
                h = 2 * ip + par

                @pl.when(h < my_slabs)
                def _(h=h, par=par):
                    do_slab(h, par)
            return carry

        lax.fori_loop(0, (max(S0, S1) + 1) // 2, slab_pair, 0)
        plsc.subcore_barrier()

        @pl.when(s < NS - 1)
        def _():
            pltpu.sync_copy(acc_s.at[pl.ds(s * STR, STR)],
                            out_hbm.at[c, pl.ds(s * STR, STR)])

        last = N - (NS - 1) * STR

        @pl.when(s == NS - 1)
        def _():
            pltpu.sync_copy(acc_s.at[pl.ds((NS - 1) * STR, last)],
                            out_hbm.at[c, pl.ds((NS - 1) * STR, last)])

    return agg


@functools.cache
def _prescale_call(N, D, NL):
    R = 1024
    G = NL // R

    def body(h_ref, x_ref, dinv_ref, g_ref):
        ones = jnp.ones((NC * NS, 1), jnp.float32)
        deg = lax.dot_general(h_ref[...], ones, (((0,), (0,)), ((), ())),
                              preferred_element_type=jnp.float32,
                              precision=lax.Precision.HIGHEST) + 1.0
        dv = lax.rsqrt(deg)
        dinv_ref[...] = dv
        g_ref[...] = dv * x_ref[...]

    return pl.pallas_call(
        body,
        grid=(G,),
        in_specs=[
            pl.BlockSpec((NC * NS, R), lambda i: (0, i)),
            pl.BlockSpec((R, D), lambda i: (i, 0)),
        ],
        out_specs=[
            pl.BlockSpec((R, 1), lambda i: (i, 0)),
            pl.BlockSpec((R, D), lambda i: (i, 0)),
        ],
        out_shape=[
            jax.ShapeDtypeStruct((NL, 1), jnp.float32),
            jax.ShapeDtypeStruct((N, D), jnp.float32),
        ],
    )


@functools.cache
def _finalize_call(N, D):
    R = 400
    G = N // R

    def body(x_ref, g_ref, acc_ref, dv_ref, w_ref, b_ref, o_ref):
        S = (acc_ref[0, :, :] + acc_ref[1, :, :] + g_ref[...]) * dv_ref[...]
        T = jnp.dot(S, w_ref[...], preferred_element_type=jnp.float32,
                    precision=lax.Precision.HIGHEST)
        o_ref[...] = x_ref[...] + jnp.maximum(T + b_ref[...], 0.0)

    return pl.pallas_call(
        body,
        grid=(G,),
        in_specs=[
            pl.BlockSpec((R, D), lambda i: (i, 0)),
            pl.BlockSpec((R, D), lambda i: (i, 0)),
            pl.BlockSpec((NC, R, D), lambda i: (0, i, 0)),
            pl.BlockSpec((R, 1), lambda i: (i, 0)),
            pl.BlockSpec((D, D), lambda i: (0, 0)),
            pl.BlockSpec((1, D), lambda i: (0, 0)),
        ],
        out_specs=pl.BlockSpec((R, D), lambda i: (i, 0)),
        out_shape=jax.ShapeDtypeStruct((N, D), jnp.float32),
    )


@jax.jit
def kernel(x, edge_index, W, b):
    N, D = x.shape
    E = edge_index.shape[1]
    src = edge_index[0]
    dst = edge_index[1]

    NL = 10240
    hist = _hist_kernel(E, N, NL)(dst)
    dinv, g = _prescale_call(N, D, NL)(hist, x)

    n_slab = NS * (S0 + S1)
    pad = n_slab * HC * K - E
    assert pad >= 0
    src_p = jnp.concatenate([src, jnp.zeros((pad,), src.dtype)])
    dst_p = jnp.concatenate([dst, jnp.full((pad,), N, dst.dtype)])
    zeros = jnp.zeros((10240 // NS, D), jnp.float32)
    acc = _agg_kernel(n_slab, HC, N, D)(
        g, zeros, src_p.reshape(n_slab, HC, K),
        dst_p.reshape(n_slab, HC, K))
    return _finalize_call(N, D)(x, g, acc, dinv, W, b.reshape(1, D))
